# Initial kernel scaffold; baseline (speedup 1.0000x reference)
import jax
import jax.numpy as jnp
from jax.experimental import pallas as pl


def kernel(position_matrix, channel_matrix, W):
    raise NotImplementedError("write your pallas kernel here")


# speedup vs baseline: 1.2547x; 1.2547x over previous
import functools

import jax
import jax.numpy as jnp
from jax.experimental import pallas as pl

N_PTS = 10000
K_NN = 16
C_IN = 128
C_OUT = 128
N_BASIS = 27

ROW_BLK = 400
AGG_BLK = 200


def _knn_body(pos_ref, post_ref, idx_ref):
    i = pl.program_id(0)
    post = post_ref[...]
    sq = jnp.sum(post * post, axis=0, keepdims=True)
    rows = pos_ref[pl.ds(i * ROW_BLK, ROW_BLK), :]
    sqr = jnp.sum(rows * rows, axis=1, keepdims=True)
    pp = jax.lax.dot_general(
        rows, post, (((1,), (0,)), ((), ())),
        preferred_element_type=jnp.float32)
    d2 = sqr + sq - 2.0 * pp
    iota = jax.lax.broadcasted_iota(jnp.int32, d2.shape, 1)
    big = jnp.int32(1 << 30)
    for t in range(K_NN):
        m = jnp.min(d2, axis=1, keepdims=True)
        cand = jnp.where(d2 == m, iota, big)
        j = jnp.min(cand, axis=1, keepdims=True)
        idx_ref[:, t:t + 1] = j
        d2 = jnp.where(iota == j, jnp.inf, d2)


def _knn(pos):
    post = pos.T
    return pl.pallas_call(
        _knn_body,
        grid=(N_PTS // ROW_BLK,),
        in_specs=[
            pl.BlockSpec((N_PTS, 3), lambda i: (0, 0)),
            pl.BlockSpec((3, N_PTS), lambda i: (0, 0)),
        ],
        out_specs=pl.BlockSpec((ROW_BLK, K_NN), lambda i: (i, 0)),
        out_shape=jax.ShapeDtypeStruct((N_PTS, K_NN), jnp.int32),
    )(pos, post)


def _agg_body(npos_ref, eig_ref, feat_ref, w_ref, cen_ref, out_ref):
    x = npos_ref[:, 0:K_NN]
    y = npos_ref[:, K_NN:2 * K_NN]
    z = npos_ref[:, 2 * K_NN:3 * K_NN]
    cx = jnp.mean(x, axis=1, keepdims=True)
    cy = jnp.mean(y, axis=1, keepdims=True)
    cz = jnp.mean(z, axis=1, keepdims=True)
    cen_ref[...] = jnp.concatenate([cx, cy, cz], axis=1)
    b2f = lambda v: v.astype(jnp.bfloat16).astype(jnp.float32)
    lx = b2f(x - cx)
    ly = b2f(y - cy)
    lz = b2f(z - cz)
    e = [b2f(eig_ref[:, k:k + 1]) for k in range(9)]
    px = lx * e[0] + ly * e[3] + lz * e[6]
    py = lx * e[1] + ly * e[4] + lz * e[7]
    pz = lx * e[2] + ly * e[5] + lz * e[8]
    r = jnp.sqrt(px * px + py * py + pz * pz + 1e-12)
    ct = jnp.clip(pz / r, -1.0 + 1e-6, 1.0 - 1e-6)
    xv = px + 1e-12
    rho = jnp.sqrt(xv * xv + py * py)
    cp = jnp.where(rho > 0.0, xv / rho, 1.0)
    one = jnp.ones_like(r)
    rad = [one, r, r * r]
    ang1 = [one, ct, 2.0 * ct * ct - 1.0]
    ang2 = [one, cp, 2.0 * cp * cp - 1.0]
    feats = feat_ref[...]
    fk = [b2f(feats[:, k * C_IN:(k + 1) * C_IN]) for k in range(K_NN)]
    g_cols = []
    for n in range(3):
        for l in range(3):
            for m in range(3):
                bas = b2f(rad[n] * ang1[l] * ang2[m])
                acc = bas[:, 0:1] * fk[0]
                for k in range(1, K_NN):
                    acc = acc + bas[:, k:k + 1] * fk[k]
                g_cols.append(acc)
    g = jnp.concatenate(g_cols, axis=1)
    out_ref[...] = jnp.dot(
        g.astype(jnp.bfloat16), w_ref[...].astype(jnp.bfloat16),
        preferred_element_type=jnp.float32)


def _aggregate(npos48, eig9, featg, wmat):
    return pl.pallas_call(
        _agg_body,
        grid=(N_PTS // AGG_BLK,),
        in_specs=[
            pl.BlockSpec((AGG_BLK, 3 * K_NN), lambda i: (i, 0)),
            pl.BlockSpec((AGG_BLK, 9), lambda i: (i, 0)),
            pl.BlockSpec((AGG_BLK, K_NN * C_IN), lambda i: (i, 0)),
            pl.BlockSpec((N_BASIS * C_IN, C_OUT), lambda i: (0, 0)),
        ],
        out_specs=[
            pl.BlockSpec((AGG_BLK, 3), lambda i: (i, 0)),
            pl.BlockSpec((AGG_BLK, C_OUT), lambda i: (i, 0)),
        ],
        out_shape=[
            jax.ShapeDtypeStruct((N_PTS, 3), jnp.float32),
            jax.ShapeDtypeStruct((N_PTS, C_OUT), jnp.float32),
        ],
    )(npos48, eig9, featg, wmat)


def kernel(position_matrix, channel_matrix, W):
    idx = _knn(position_matrix)
    npos = position_matrix[idx]
    centers0 = jnp.mean(npos, axis=1)
    local0 = npos - centers0[:, None, :]
    cov = jnp.einsum('pki,pkj->pij', local0, local0) / float(K_NN)
    _, eigvec = jnp.linalg.eigh(cov)
    featg = channel_matrix[idx].reshape(N_PTS, K_NN * C_IN)
    npos48 = npos.transpose(0, 2, 1).reshape(N_PTS, 3 * K_NN)
    eig9 = eigvec.reshape(N_PTS, 9)
    wmat = W.transpose(2, 1, 0).reshape(N_BASIS * C_IN, C_OUT)
    centers, out = _aggregate(npos48, eig9, featg, wmat)
    return centers, out


# speedup vs baseline: 11.3929x; 9.0804x over previous
import functools

import jax
import jax.numpy as jnp
from jax.experimental import pallas as pl

N_PTS = 10000
K_NN = 16
C_IN = 128
C_OUT = 128
N_BASIS = 27

ROW_BLK = 400
AGG_BLK = 200


def _knn_body(pos_ref, post_ref, idx_ref):
    i = pl.program_id(0)
    post = post_ref[...]
    sq = jnp.sum(post * post, axis=0, keepdims=True)
    rows = pos_ref[pl.ds(i * ROW_BLK, ROW_BLK), :]
    sqr = jnp.sum(rows * rows, axis=1, keepdims=True)
    pp = jax.lax.dot_general(
        rows, post, (((1,), (0,)), ((), ())),
        preferred_element_type=jnp.float32)
    d2 = sqr + sq - 2.0 * pp
    iota = jax.lax.broadcasted_iota(jnp.int32, d2.shape, 1)
    big = jnp.int32(1 << 30)
    for t in range(K_NN):
        m = jnp.min(d2, axis=1, keepdims=True)
        cand = jnp.where(d2 == m, iota, big)
        j = jnp.min(cand, axis=1, keepdims=True)
        idx_ref[:, t:t + 1] = j
        d2 = jnp.where(iota == j, jnp.inf, d2)


def _knn(pos):
    post = pos.T
    return pl.pallas_call(
        _knn_body,
        grid=(N_PTS // ROW_BLK,),
        in_specs=[
            pl.BlockSpec((N_PTS, 3), lambda i: (0, 0)),
            pl.BlockSpec((3, N_PTS), lambda i: (0, 0)),
        ],
        out_specs=pl.BlockSpec((ROW_BLK, K_NN), lambda i: (i, 0)),
        out_shape=jax.ShapeDtypeStruct((N_PTS, K_NN), jnp.int32),
    )(pos, post)


def _eigh3_body(cov_ref, v_ref):
    one = jnp.ones((1, N_PTS), jnp.float32)
    zero = jnp.zeros((1, N_PTS), jnp.float32)
    a = {(0, 0): cov_ref[0:1, :], (0, 1): cov_ref[1:2, :],
         (0, 2): cov_ref[2:3, :], (1, 1): cov_ref[3:4, :],
         (1, 2): cov_ref[4:5, :], (2, 2): cov_ref[5:6, :]}
    v = {(i, j): (one if i == j else zero) for i in range(3) for j in range(3)}
    for _ in range(12):
        for (p, q) in ((0, 2), (1, 2), (0, 1)):
            r = 3 - p - q
            app, aqq, apq = a[(p, p)], a[(q, q)], a[(p, q)]
            arp = a[(min(r, p), max(r, p))]
            arq = a[(min(r, q), max(r, q))]
            tau = (aqq - app) / (2.0 * apq)
            sgn = jnp.where(tau >= 0.0, 1.0, -1.0)
            t = sgn / (jnp.abs(tau) + jnp.sqrt(1.0 + tau * tau))
            c = 1.0 / jnp.sqrt(1.0 + t * t)
            s = t * c
            iszero = apq == 0.0
            c = jnp.where(iszero, 1.0, c)
            s = jnp.where(iszero, 0.0, s)
            cc, ss, cs = c * c, s * s, c * s
            a[(p, p)] = cc * app - 2.0 * cs * apq + ss * aqq
            a[(q, q)] = ss * app + 2.0 * cs * apq + cc * aqq
            a[(p, q)] = cs * (app - aqq) + (cc - ss) * apq
            a[(min(r, p), max(r, p))] = c * arp - s * arq
            a[(min(r, q), max(r, q))] = s * arp + c * arq
            for i in range(3):
                vp, vq = v[(i, p)], v[(i, q)]
                v[(i, p)] = c * vp - s * vq
                v[(i, q)] = s * vp + c * vq
    w = [a[(0, 0)], a[(1, 1)], a[(2, 2)]]
    cols = [[v[(i, j)] for i in range(3)] for j in range(3)]
    for (i, j) in ((0, 1), (1, 2), (0, 1)):
        sw = w[j] < w[i]
        w[i], w[j] = (jnp.where(sw, w[j], w[i]), jnp.where(sw, w[i], w[j]))
        ci = [jnp.where(sw, cols[j][k], cols[i][k]) for k in range(3)]
        cj = [jnp.where(sw, cols[i][k], cols[j][k]) for k in range(3)]
        cols[i], cols[j] = ci, cj
    for i in range(3):
        for j in range(3):
            v_ref[3 * i + j:3 * i + j + 1, :] = cols[j][i]


def _eigh3(cov6):
    return pl.pallas_call(
        _eigh3_body,
        grid=(1,),
        in_specs=[pl.BlockSpec((6, N_PTS), lambda i: (0, 0))],
        out_specs=pl.BlockSpec((9, N_PTS), lambda i: (0, 0)),
        out_shape=jax.ShapeDtypeStruct((9, N_PTS), jnp.float32),
    )(cov6)


def _agg_body(npos_ref, eig_ref, feat_ref, w_ref, cen_ref, out_ref):
    x = npos_ref[:, 0:K_NN]
    y = npos_ref[:, K_NN:2 * K_NN]
    z = npos_ref[:, 2 * K_NN:3 * K_NN]
    cx = jnp.mean(x, axis=1, keepdims=True)
    cy = jnp.mean(y, axis=1, keepdims=True)
    cz = jnp.mean(z, axis=1, keepdims=True)
    cen_ref[...] = jnp.concatenate([cx, cy, cz], axis=1)
    b2f = lambda v: v.astype(jnp.bfloat16).astype(jnp.float32)
    lx = b2f(x - cx)
    ly = b2f(y - cy)
    lz = b2f(z - cz)
    e = [b2f(eig_ref[:, k:k + 1]) for k in range(9)]
    px = lx * e[0] + ly * e[3] + lz * e[6]
    py = lx * e[1] + ly * e[4] + lz * e[7]
    pz = lx * e[2] + ly * e[5] + lz * e[8]
    r = jnp.sqrt(px * px + py * py + pz * pz + 1e-12)
    ct = jnp.clip(pz / r, -1.0 + 1e-6, 1.0 - 1e-6)
    xv = px + 1e-12
    rho = jnp.sqrt(xv * xv + py * py)
    cp = jnp.where(rho > 0.0, xv / rho, 1.0)
    one = jnp.ones_like(r)
    rad = [one, r, r * r]
    ang1 = [one, ct, 2.0 * ct * ct - 1.0]
    ang2 = [one, cp, 2.0 * cp * cp - 1.0]
    feats = feat_ref[...]
    fk = [b2f(feats[:, k * C_IN:(k + 1) * C_IN]) for k in range(K_NN)]
    g_cols = []
    for n in range(3):
        for l in range(3):
            for m in range(3):
                bas = b2f(rad[n] * ang1[l] * ang2[m])
                acc = bas[:, 0:1] * fk[0]
                for k in range(1, K_NN):
                    acc = acc + bas[:, k:k + 1] * fk[k]
                g_cols.append(acc)
    g = jnp.concatenate(g_cols, axis=1)
    out_ref[...] = jnp.dot(
        g.astype(jnp.bfloat16), w_ref[...].astype(jnp.bfloat16),
        preferred_element_type=jnp.float32)


def _aggregate(npos48, eig9, featg, wmat):
    return pl.pallas_call(
        _agg_body,
        grid=(N_PTS // AGG_BLK,),
        in_specs=[
            pl.BlockSpec((AGG_BLK, 3 * K_NN), lambda i: (i, 0)),
            pl.BlockSpec((AGG_BLK, 9), lambda i: (i, 0)),
            pl.BlockSpec((AGG_BLK, K_NN * C_IN), lambda i: (i, 0)),
            pl.BlockSpec((N_BASIS * C_IN, C_OUT), lambda i: (0, 0)),
        ],
        out_specs=[
            pl.BlockSpec((AGG_BLK, 3), lambda i: (i, 0)),
            pl.BlockSpec((AGG_BLK, C_OUT), lambda i: (i, 0)),
        ],
        out_shape=[
            jax.ShapeDtypeStruct((N_PTS, 3), jnp.float32),
            jax.ShapeDtypeStruct((N_PTS, C_OUT), jnp.float32),
        ],
    )(npos48, eig9, featg, wmat)


def kernel(position_matrix, channel_matrix, W):
    idx = _knn(position_matrix)
    npos = position_matrix[idx]
    centers0 = jnp.mean(npos, axis=1)
    local0 = npos - centers0[:, None, :]
    cov = jnp.einsum('pki,pkj->pij', local0, local0) / float(K_NN)
    cov6 = jnp.stack([cov[:, 0, 0], cov[:, 0, 1], cov[:, 0, 2],
                      cov[:, 1, 1], cov[:, 1, 2], cov[:, 2, 2]], axis=0)
    eig9 = _eigh3(cov6).T
    featg = channel_matrix[idx].reshape(N_PTS, K_NN * C_IN)
    npos48 = npos.transpose(0, 2, 1).reshape(N_PTS, 3 * K_NN)
    wmat = W.transpose(2, 1, 0).reshape(N_BASIS * C_IN, C_OUT)
    centers, out = _aggregate(npos48, eig9, featg, wmat)
    return centers, out


# speedup vs baseline: 11.3951x; 1.0002x over previous
import functools

import jax
import jax.numpy as jnp
from jax.experimental import pallas as pl

N_PTS = 10000
K_NN = 16
C_IN = 128
C_OUT = 128
N_BASIS = 27

ROW_BLK = 400
AGG_BLK = 200


def _knn_body(pos_ref, post_ref, sqc_ref, sqr_ref, idx_ref):
    i = pl.program_id(0)
    post = post_ref[...]
    sq = sqc_ref[...]
    rows = pos_ref[pl.ds(i * ROW_BLK, ROW_BLK), :]
    sqr = sqr_ref[pl.ds(i * ROW_BLK, ROW_BLK), :]
    pp = jax.lax.dot_general(
        rows, post, (((1,), (0,)), ((), ())),
        preferred_element_type=jnp.float32)
    d2 = sqr + sq - 2.0 * pp
    iota = jax.lax.broadcasted_iota(jnp.int32, d2.shape, 1)
    big = jnp.int32(1 << 30)
    for t in range(K_NN):
        m = jnp.min(d2, axis=1, keepdims=True)
        cand = jnp.where(d2 == m, iota, big)
        j = jnp.min(cand, axis=1, keepdims=True)
        idx_ref[:, t:t + 1] = j
        d2 = jnp.where(iota == j, jnp.inf, d2)


def _knn(pos):
    post = pos.T
    sq = jnp.sum(pos * pos, axis=1)
    return pl.pallas_call(
        _knn_body,
        grid=(N_PTS // ROW_BLK,),
        in_specs=[
            pl.BlockSpec((N_PTS, 3), lambda i: (0, 0)),
            pl.BlockSpec((3, N_PTS), lambda i: (0, 0)),
            pl.BlockSpec((1, N_PTS), lambda i: (0, 0)),
            pl.BlockSpec((N_PTS, 1), lambda i: (0, 0)),
        ],
        out_specs=pl.BlockSpec((ROW_BLK, K_NN), lambda i: (i, 0)),
        out_shape=jax.ShapeDtypeStruct((N_PTS, K_NN), jnp.int32),
    )(pos, post, sq.reshape(1, N_PTS), sq.reshape(N_PTS, 1))


def _eigh3_body(cov_ref, v_ref):
    one = jnp.ones((1, N_PTS), jnp.float32)
    zero = jnp.zeros((1, N_PTS), jnp.float32)
    a = {(0, 0): cov_ref[0:1, :], (0, 1): cov_ref[1:2, :],
         (0, 2): cov_ref[2:3, :], (1, 1): cov_ref[3:4, :],
         (1, 2): cov_ref[4:5, :], (2, 2): cov_ref[5:6, :]}
    v = {(i, j): (one if i == j else zero) for i in range(3) for j in range(3)}
    for _ in range(12):
        for (p, q) in ((0, 2), (1, 2), (0, 1)):
            r = 3 - p - q
            app, aqq, apq = a[(p, p)], a[(q, q)], a[(p, q)]
            arp = a[(min(r, p), max(r, p))]
            arq = a[(min(r, q), max(r, q))]
            tau = (aqq - app) / (2.0 * apq)
            sgn = jnp.where(tau >= 0.0, 1.0, -1.0)
            t = sgn / (jnp.abs(tau) + jnp.sqrt(1.0 + tau * tau))
            c = 1.0 / jnp.sqrt(1.0 + t * t)
            s = t * c
            iszero = apq == 0.0
            c = jnp.where(iszero, 1.0, c)
            s = jnp.where(iszero, 0.0, s)
            cc, ss, cs = c * c, s * s, c * s
            a[(p, p)] = cc * app - 2.0 * cs * apq + ss * aqq
            a[(q, q)] = ss * app + 2.0 * cs * apq + cc * aqq
            a[(p, q)] = cs * (app - aqq) + (cc - ss) * apq
            a[(min(r, p), max(r, p))] = c * arp - s * arq
            a[(min(r, q), max(r, q))] = s * arp + c * arq
            for i in range(3):
                vp, vq = v[(i, p)], v[(i, q)]
                v[(i, p)] = c * vp - s * vq
                v[(i, q)] = s * vp + c * vq
    w = [a[(0, 0)], a[(1, 1)], a[(2, 2)]]
    cols = [[v[(i, j)] for i in range(3)] for j in range(3)]
    for (i, j) in ((0, 1), (1, 2), (0, 1)):
        sw = w[j] < w[i]
        w[i], w[j] = (jnp.where(sw, w[j], w[i]), jnp.where(sw, w[i], w[j]))
        ci = [jnp.where(sw, cols[j][k], cols[i][k]) for k in range(3)]
        cj = [jnp.where(sw, cols[i][k], cols[j][k]) for k in range(3)]
        cols[i], cols[j] = ci, cj
    for i in range(3):
        for j in range(3):
            v_ref[3 * i + j:3 * i + j + 1, :] = cols[j][i]


def _eigh3(cov6):
    return pl.pallas_call(
        _eigh3_body,
        grid=(1,),
        in_specs=[pl.BlockSpec((6, N_PTS), lambda i: (0, 0))],
        out_specs=pl.BlockSpec((9, N_PTS), lambda i: (0, 0)),
        out_shape=jax.ShapeDtypeStruct((9, N_PTS), jnp.float32),
    )(cov6)


def _agg_body(npos_ref, eig_ref, feat_ref, w_ref, cen_ref, out_ref):
    x = npos_ref[:, 0:K_NN]
    y = npos_ref[:, K_NN:2 * K_NN]
    z = npos_ref[:, 2 * K_NN:3 * K_NN]
    cx = jnp.mean(x, axis=1, keepdims=True)
    cy = jnp.mean(y, axis=1, keepdims=True)
    cz = jnp.mean(z, axis=1, keepdims=True)
    cen_ref[...] = jnp.concatenate([cx, cy, cz], axis=1)
    b2f = lambda v: v.astype(jnp.bfloat16).astype(jnp.float32)
    lx = b2f(x - cx)
    ly = b2f(y - cy)
    lz = b2f(z - cz)
    e = [b2f(eig_ref[:, k:k + 1]) for k in range(9)]
    px = lx * e[0] + ly * e[3] + lz * e[6]
    py = lx * e[1] + ly * e[4] + lz * e[7]
    pz = lx * e[2] + ly * e[5] + lz * e[8]
    r = jnp.sqrt(px * px + py * py + pz * pz + 1e-12)
    ct = jnp.clip(pz / r, -1.0 + 1e-6, 1.0 - 1e-6)
    xv = px + 1e-12
    rho = jnp.sqrt(xv * xv + py * py)
    cp = jnp.where(rho > 0.0, xv / rho, 1.0)
    one = jnp.ones_like(r)
    rad = [one, r, r * r]
    ang1 = [one, ct, 2.0 * ct * ct - 1.0]
    ang2 = [one, cp, 2.0 * cp * cp - 1.0]
    feats = feat_ref[...]
    fk = [b2f(feats[:, k * C_IN:(k + 1) * C_IN]) for k in range(K_NN)]
    g_cols = []
    for n in range(3):
        for l in range(3):
            for m in range(3):
                bas = b2f(rad[n] * ang1[l] * ang2[m])
                acc = bas[:, 0:1] * fk[0]
                for k in range(1, K_NN):
                    acc = acc + bas[:, k:k + 1] * fk[k]
                g_cols.append(acc)
    g = jnp.concatenate(g_cols, axis=1)
    out_ref[...] = jnp.dot(
        g.astype(jnp.bfloat16), w_ref[...].astype(jnp.bfloat16),
        preferred_element_type=jnp.float32)


def _aggregate(npos48, eig9, featg, wmat):
    return pl.pallas_call(
        _agg_body,
        grid=(N_PTS // AGG_BLK,),
        in_specs=[
            pl.BlockSpec((AGG_BLK, 3 * K_NN), lambda i: (i, 0)),
            pl.BlockSpec((AGG_BLK, 9), lambda i: (i, 0)),
            pl.BlockSpec((AGG_BLK, K_NN * C_IN), lambda i: (i, 0)),
            pl.BlockSpec((N_BASIS * C_IN, C_OUT), lambda i: (0, 0)),
        ],
        out_specs=[
            pl.BlockSpec((AGG_BLK, 3), lambda i: (i, 0)),
            pl.BlockSpec((AGG_BLK, C_OUT), lambda i: (i, 0)),
        ],
        out_shape=[
            jax.ShapeDtypeStruct((N_PTS, 3), jnp.float32),
            jax.ShapeDtypeStruct((N_PTS, C_OUT), jnp.float32),
        ],
    )(npos48, eig9, featg, wmat)


def kernel(position_matrix, channel_matrix, W):
    idx = _knn(position_matrix)
    npos = position_matrix[idx]
    centers0 = jnp.mean(npos, axis=1)
    local0 = npos - centers0[:, None, :]
    cov = jnp.einsum('pki,pkj->pij', local0, local0) / float(K_NN)
    cov6 = jnp.stack([cov[:, 0, 0], cov[:, 0, 1], cov[:, 0, 2],
                      cov[:, 1, 1], cov[:, 1, 2], cov[:, 2, 2]], axis=0)
    eig9 = _eigh3(cov6).T
    featg = channel_matrix[idx].reshape(N_PTS, K_NN * C_IN)
    npos48 = npos.transpose(0, 2, 1).reshape(N_PTS, 3 * K_NN)
    wmat = W.transpose(2, 1, 0).reshape(N_BASIS * C_IN, C_OUT)
    centers, out = _aggregate(npos48, eig9, featg, wmat)
    return centers, out


# speedup vs baseline: 13.7566x; 1.2072x over previous
import functools

import jax
import jax.numpy as jnp
from jax import lax
from jax.experimental import pallas as pl
from jax.experimental.pallas import tpu as pltpu
from jax.experimental.pallas import tpu_sc as plsc

N_PTS = 10000
K_NN = 16
C_IN = 128
C_OUT = 128
N_BASIS = 27

ROW_BLK = 400
AGG_BLK = 200


def _knn_body(pos_ref, post_ref, sqc_ref, sqr_ref, idx_ref):
    i = pl.program_id(0)
    post = post_ref[...]
    sq = sqc_ref[...]
    rows = pos_ref[pl.ds(i * ROW_BLK, ROW_BLK), :]
    sqr = sqr_ref[pl.ds(i * ROW_BLK, ROW_BLK), :]
    pp = jax.lax.dot_general(
        rows, post, (((1,), (0,)), ((), ())),
        preferred_element_type=jnp.float32)
    d2 = sqr + sq - 2.0 * pp
    iota = jax.lax.broadcasted_iota(jnp.int32, d2.shape, 1)
    big = jnp.int32(1 << 30)
    for t in range(K_NN):
        m = jnp.min(d2, axis=1, keepdims=True)
        cand = jnp.where(d2 == m, iota, big)
        j = jnp.min(cand, axis=1, keepdims=True)
        idx_ref[:, t:t + 1] = j
        d2 = jnp.where(iota == j, jnp.inf, d2)


def _knn(pos):
    post = pos.T
    sq = jnp.sum(pos * pos, axis=1)
    return pl.pallas_call(
        _knn_body,
        grid=(N_PTS // ROW_BLK,),
        in_specs=[
            pl.BlockSpec((N_PTS, 3), lambda i: (0, 0)),
            pl.BlockSpec((3, N_PTS), lambda i: (0, 0)),
            pl.BlockSpec((1, N_PTS), lambda i: (0, 0)),
            pl.BlockSpec((N_PTS, 1), lambda i: (0, 0)),
        ],
        out_specs=pl.BlockSpec((ROW_BLK, K_NN), lambda i: (i, 0)),
        out_shape=jax.ShapeDtypeStruct((N_PTS, K_NN), jnp.int32),
    )(pos, post, sq.reshape(1, N_PTS), sq.reshape(N_PTS, 1))


_SC_INFO = plsc.get_sparse_core_info()
_NW = _SC_INFO.num_cores * _SC_INFO.num_subcores
_GATHER_B = N_PTS * K_NN
_GATHER_CHUNK = 200


def _make_sc_gather(depth):
    b_per_w = _GATHER_B // _NW
    n_chunks = b_per_w // _GATHER_CHUNK
    mesh = plsc.VectorSubcoreMesh(core_axis_name="c", subcore_axis_name="s")

    @functools.partial(
        pl.kernel, mesh=mesh,
        out_type=jax.ShapeDtypeStruct((_GATHER_B, depth), jnp.float32),
        scratch_types=[
            pltpu.VMEM((_GATHER_CHUNK,), jnp.int32),
            pltpu.VMEM((_GATHER_CHUNK, depth), jnp.float32),
            pltpu.SemaphoreType.DMA,
        ],
    )
    def gather(table_hbm, idx_hbm, out_hbm, idx_v, rows_v, sem):
        wid = lax.axis_index("s") * _SC_INFO.num_cores + lax.axis_index("c")
        base = wid * b_per_w

        def body(t):
            off = base + t * _GATHER_CHUNK
            pltpu.sync_copy(idx_hbm.at[pl.ds(off, _GATHER_CHUNK)], idx_v)
            pltpu.async_copy(table_hbm.at[idx_v], rows_v, sem).wait()
            pltpu.sync_copy(rows_v, out_hbm.at[pl.ds(off, _GATHER_CHUNK)])

        pl.loop(0, n_chunks)(body)

    return gather


_sc_gather_rows = _make_sc_gather(C_IN)


def _eigh3_body(cov_ref, v_ref):
    one = jnp.ones((1, N_PTS), jnp.float32)
    zero = jnp.zeros((1, N_PTS), jnp.float32)
    a = {(0, 0): cov_ref[0:1, :], (0, 1): cov_ref[1:2, :],
         (0, 2): cov_ref[2:3, :], (1, 1): cov_ref[3:4, :],
         (1, 2): cov_ref[4:5, :], (2, 2): cov_ref[5:6, :]}
    v = {(i, j): (one if i == j else zero) for i in range(3) for j in range(3)}
    for _ in range(12):
        for (p, q) in ((0, 2), (1, 2), (0, 1)):
            r = 3 - p - q
            app, aqq, apq = a[(p, p)], a[(q, q)], a[(p, q)]
            arp = a[(min(r, p), max(r, p))]
            arq = a[(min(r, q), max(r, q))]
            tau = (aqq - app) / (2.0 * apq)
            sgn = jnp.where(tau >= 0.0, 1.0, -1.0)
            t = sgn / (jnp.abs(tau) + jnp.sqrt(1.0 + tau * tau))
            c = 1.0 / jnp.sqrt(1.0 + t * t)
            s = t * c
            iszero = apq == 0.0
            c = jnp.where(iszero, 1.0, c)
            s = jnp.where(iszero, 0.0, s)
            cc, ss, cs = c * c, s * s, c * s
            a[(p, p)] = cc * app - 2.0 * cs * apq + ss * aqq
            a[(q, q)] = ss * app + 2.0 * cs * apq + cc * aqq
            a[(p, q)] = cs * (app - aqq) + (cc - ss) * apq
            a[(min(r, p), max(r, p))] = c * arp - s * arq
            a[(min(r, q), max(r, q))] = s * arp + c * arq
            for i in range(3):
                vp, vq = v[(i, p)], v[(i, q)]
                v[(i, p)] = c * vp - s * vq
                v[(i, q)] = s * vp + c * vq
    w = [a[(0, 0)], a[(1, 1)], a[(2, 2)]]
    cols = [[v[(i, j)] for i in range(3)] for j in range(3)]
    for (i, j) in ((0, 1), (1, 2), (0, 1)):
        sw = w[j] < w[i]
        w[i], w[j] = (jnp.where(sw, w[j], w[i]), jnp.where(sw, w[i], w[j]))
        ci = [jnp.where(sw, cols[j][k], cols[i][k]) for k in range(3)]
        cj = [jnp.where(sw, cols[i][k], cols[j][k]) for k in range(3)]
        cols[i], cols[j] = ci, cj
    for i in range(3):
        for j in range(3):
            v_ref[3 * i + j:3 * i + j + 1, :] = cols[j][i]


def _eigh3(cov6):
    return pl.pallas_call(
        _eigh3_body,
        grid=(1,),
        in_specs=[pl.BlockSpec((6, N_PTS), lambda i: (0, 0))],
        out_specs=pl.BlockSpec((9, N_PTS), lambda i: (0, 0)),
        out_shape=jax.ShapeDtypeStruct((9, N_PTS), jnp.float32),
    )(cov6)


def _agg_body(npos_ref, eig_ref, feat_ref, w_ref, cen_ref, out_ref):
    x = npos_ref[:, 0:K_NN]
    y = npos_ref[:, K_NN:2 * K_NN]
    z = npos_ref[:, 2 * K_NN:3 * K_NN]
    cx = jnp.mean(x, axis=1, keepdims=True)
    cy = jnp.mean(y, axis=1, keepdims=True)
    cz = jnp.mean(z, axis=1, keepdims=True)
    cen_ref[...] = jnp.concatenate([cx, cy, cz], axis=1)
    b2f = lambda v: v.astype(jnp.bfloat16).astype(jnp.float32)
    lx = b2f(x - cx)
    ly = b2f(y - cy)
    lz = b2f(z - cz)
    e = [b2f(eig_ref[:, k:k + 1]) for k in range(9)]
    px = lx * e[0] + ly * e[3] + lz * e[6]
    py = lx * e[1] + ly * e[4] + lz * e[7]
    pz = lx * e[2] + ly * e[5] + lz * e[8]
    r = jnp.sqrt(px * px + py * py + pz * pz + 1e-12)
    ct = jnp.clip(pz / r, -1.0 + 1e-6, 1.0 - 1e-6)
    xv = px + 1e-12
    rho = jnp.sqrt(xv * xv + py * py)
    cp = jnp.where(rho > 0.0, xv / rho, 1.0)
    one = jnp.ones_like(r)
    rad = [one, r, r * r]
    ang1 = [one, ct, 2.0 * ct * ct - 1.0]
    ang2 = [one, cp, 2.0 * cp * cp - 1.0]
    feats = feat_ref[...]
    fk = [b2f(feats[:, k * C_IN:(k + 1) * C_IN]) for k in range(K_NN)]
    g_cols = []
    for n in range(3):
        for l in range(3):
            for m in range(3):
                bas = b2f(rad[n] * ang1[l] * ang2[m])
                acc = bas[:, 0:1] * fk[0]
                for k in range(1, K_NN):
                    acc = acc + bas[:, k:k + 1] * fk[k]
                g_cols.append(acc)
    g = jnp.concatenate(g_cols, axis=1)
    out_ref[...] = jnp.dot(
        g.astype(jnp.bfloat16), w_ref[...].astype(jnp.bfloat16),
        preferred_element_type=jnp.float32)


def _aggregate(npos48, eig9, featg, wmat):
    return pl.pallas_call(
        _agg_body,
        grid=(N_PTS // AGG_BLK,),
        in_specs=[
            pl.BlockSpec((AGG_BLK, 3 * K_NN), lambda i: (i, 0)),
            pl.BlockSpec((AGG_BLK, 9), lambda i: (i, 0)),
            pl.BlockSpec((AGG_BLK, K_NN * C_IN), lambda i: (i, 0)),
            pl.BlockSpec((N_BASIS * C_IN, C_OUT), lambda i: (0, 0)),
        ],
        out_specs=[
            pl.BlockSpec((AGG_BLK, 3), lambda i: (i, 0)),
            pl.BlockSpec((AGG_BLK, C_OUT), lambda i: (i, 0)),
        ],
        out_shape=[
            jax.ShapeDtypeStruct((N_PTS, 3), jnp.float32),
            jax.ShapeDtypeStruct((N_PTS, C_OUT), jnp.float32),
        ],
    )(npos48, eig9, featg, wmat)


def kernel(position_matrix, channel_matrix, W):
    idx = _knn(position_matrix)
    idxf = idx.reshape(_GATHER_B)
    pos128 = jnp.pad(position_matrix, ((0, 0), (0, C_IN - 3)))
    npos = _sc_gather_rows(pos128, idxf).reshape(N_PTS, K_NN, C_IN)[:, :, :3]
    centers0 = jnp.mean(npos, axis=1)
    local0 = npos - centers0[:, None, :]
    cov = jnp.einsum('pki,pkj->pij', local0, local0) / float(K_NN)
    cov6 = jnp.stack([cov[:, 0, 0], cov[:, 0, 1], cov[:, 0, 2],
                      cov[:, 1, 1], cov[:, 1, 2], cov[:, 2, 2]], axis=0)
    eig9 = _eigh3(cov6).T
    featg = _sc_gather_rows(channel_matrix, idxf).reshape(N_PTS, K_NN * C_IN)
    npos48 = npos.transpose(0, 2, 1).reshape(N_PTS, 3 * K_NN)
    wmat = W.transpose(2, 1, 0).reshape(N_BASIS * C_IN, C_OUT)
    centers, out = _aggregate(npos48, eig9, featg, wmat)
    return centers, out


# speedup vs baseline: 13.7579x; 1.0001x over previous
import functools

import jax
import jax.numpy as jnp
from jax import lax
from jax.experimental import pallas as pl
from jax.experimental.pallas import tpu as pltpu
from jax.experimental.pallas import tpu_sc as plsc

N_PTS = 10000
K_NN = 16
C_IN = 128
C_OUT = 128
N_BASIS = 27

ROW_BLK = 400
AGG_BLK = 200


def _knn_body(pos_ref, post_ref, sqc_ref, sqr_ref, idx_ref):
    i = pl.program_id(0)
    post = post_ref[...]
    sq = sqc_ref[...]
    rows = pos_ref[pl.ds(i * ROW_BLK, ROW_BLK), :]
    sqr = sqr_ref[pl.ds(i * ROW_BLK, ROW_BLK), :]
    pp = jax.lax.dot_general(
        rows, post, (((1,), (0,)), ((), ())),
        preferred_element_type=jnp.float32)
    d2 = sqr + sq - 2.0 * pp
    iota = jax.lax.broadcasted_iota(jnp.int32, d2.shape, 1)
    big = jnp.int32(1 << 30)
    for t in range(K_NN):
        m = jnp.min(d2, axis=1, keepdims=True)
        cand = jnp.where(d2 == m, iota, big)
        j = jnp.min(cand, axis=1, keepdims=True)
        idx_ref[:, t:t + 1] = j
        d2 = jnp.where(iota == j, jnp.inf, d2)


def _knn(pos):
    post = pos.T
    sq = jnp.sum(pos * pos, axis=1)
    return pl.pallas_call(
        _knn_body,
        grid=(N_PTS // ROW_BLK,),
        in_specs=[
            pl.BlockSpec((N_PTS, 3), lambda i: (0, 0)),
            pl.BlockSpec((3, N_PTS), lambda i: (0, 0)),
            pl.BlockSpec((1, N_PTS), lambda i: (0, 0)),
            pl.BlockSpec((N_PTS, 1), lambda i: (0, 0)),
        ],
        out_specs=pl.BlockSpec((ROW_BLK, K_NN), lambda i: (i, 0)),
        out_shape=jax.ShapeDtypeStruct((N_PTS, K_NN), jnp.int32),
    )(pos, post, sq.reshape(1, N_PTS), sq.reshape(N_PTS, 1))


_GATHER_B = N_PTS * K_NN
_GATHER_CHUNK = 200


@functools.lru_cache(maxsize=None)
def _make_sc_gather(depth):
    info = plsc.get_sparse_core_info()
    num_cores = info.num_cores
    b_per_w = _GATHER_B // (num_cores * info.num_subcores)
    n_chunks = b_per_w // _GATHER_CHUNK
    mesh = plsc.VectorSubcoreMesh(core_axis_name="c", subcore_axis_name="s")

    @functools.partial(
        pl.kernel, mesh=mesh,
        out_type=jax.ShapeDtypeStruct((_GATHER_B, depth), jnp.float32),
        scratch_types=[
            pltpu.VMEM((_GATHER_CHUNK,), jnp.int32),
            pltpu.VMEM((_GATHER_CHUNK, depth), jnp.float32),
            pltpu.SemaphoreType.DMA,
        ],
    )
    def gather(table_hbm, idx_hbm, out_hbm, idx_v, rows_v, sem):
        wid = lax.axis_index("s") * num_cores + lax.axis_index("c")
        base = wid * b_per_w

        def body(t):
            off = base + t * _GATHER_CHUNK
            pltpu.sync_copy(idx_hbm.at[pl.ds(off, _GATHER_CHUNK)], idx_v)
            pltpu.async_copy(table_hbm.at[idx_v], rows_v, sem).wait()
            pltpu.sync_copy(rows_v, out_hbm.at[pl.ds(off, _GATHER_CHUNK)])

        pl.loop(0, n_chunks)(body)

    return gather


def _sc_gather_rows(table, idxf):
    return _make_sc_gather(C_IN)(table, idxf)


def _eigh3_body(cov_ref, v_ref):
    one = jnp.ones((1, N_PTS), jnp.float32)
    zero = jnp.zeros((1, N_PTS), jnp.float32)
    a = {(0, 0): cov_ref[0:1, :], (0, 1): cov_ref[1:2, :],
         (0, 2): cov_ref[2:3, :], (1, 1): cov_ref[3:4, :],
         (1, 2): cov_ref[4:5, :], (2, 2): cov_ref[5:6, :]}
    v = {(i, j): (one if i == j else zero) for i in range(3) for j in range(3)}
    for _ in range(12):
        for (p, q) in ((0, 2), (1, 2), (0, 1)):
            r = 3 - p - q
            app, aqq, apq = a[(p, p)], a[(q, q)], a[(p, q)]
            arp = a[(min(r, p), max(r, p))]
            arq = a[(min(r, q), max(r, q))]
            tau = (aqq - app) / (2.0 * apq)
            sgn = jnp.where(tau >= 0.0, 1.0, -1.0)
            t = sgn / (jnp.abs(tau) + jnp.sqrt(1.0 + tau * tau))
            c = 1.0 / jnp.sqrt(1.0 + t * t)
            s = t * c
            iszero = apq == 0.0
            c = jnp.where(iszero, 1.0, c)
            s = jnp.where(iszero, 0.0, s)
            cc, ss, cs = c * c, s * s, c * s
            a[(p, p)] = cc * app - 2.0 * cs * apq + ss * aqq
            a[(q, q)] = ss * app + 2.0 * cs * apq + cc * aqq
            a[(p, q)] = cs * (app - aqq) + (cc - ss) * apq
            a[(min(r, p), max(r, p))] = c * arp - s * arq
            a[(min(r, q), max(r, q))] = s * arp + c * arq
            for i in range(3):
                vp, vq = v[(i, p)], v[(i, q)]
                v[(i, p)] = c * vp - s * vq
                v[(i, q)] = s * vp + c * vq
    w = [a[(0, 0)], a[(1, 1)], a[(2, 2)]]
    cols = [[v[(i, j)] for i in range(3)] for j in range(3)]
    for (i, j) in ((0, 1), (1, 2), (0, 1)):
        sw = w[j] < w[i]
        w[i], w[j] = (jnp.where(sw, w[j], w[i]), jnp.where(sw, w[i], w[j]))
        ci = [jnp.where(sw, cols[j][k], cols[i][k]) for k in range(3)]
        cj = [jnp.where(sw, cols[i][k], cols[j][k]) for k in range(3)]
        cols[i], cols[j] = ci, cj
    for i in range(3):
        for j in range(3):
            v_ref[3 * i + j:3 * i + j + 1, :] = cols[j][i]


def _eigh3(cov6):
    return pl.pallas_call(
        _eigh3_body,
        grid=(1,),
        in_specs=[pl.BlockSpec((6, N_PTS), lambda i: (0, 0))],
        out_specs=pl.BlockSpec((9, N_PTS), lambda i: (0, 0)),
        out_shape=jax.ShapeDtypeStruct((9, N_PTS), jnp.float32),
    )(cov6)


def _agg_body(npos_ref, eig_ref, feat_ref, w_ref, cen_ref, out_ref):
    x = npos_ref[:, 0:K_NN]
    y = npos_ref[:, K_NN:2 * K_NN]
    z = npos_ref[:, 2 * K_NN:3 * K_NN]
    cx = jnp.mean(x, axis=1, keepdims=True)
    cy = jnp.mean(y, axis=1, keepdims=True)
    cz = jnp.mean(z, axis=1, keepdims=True)
    cen_ref[...] = jnp.concatenate([cx, cy, cz], axis=1)
    b2f = lambda v: v.astype(jnp.bfloat16).astype(jnp.float32)
    lx = b2f(x - cx)
    ly = b2f(y - cy)
    lz = b2f(z - cz)
    e = [b2f(eig_ref[:, k:k + 1]) for k in range(9)]
    px = lx * e[0] + ly * e[3] + lz * e[6]
    py = lx * e[1] + ly * e[4] + lz * e[7]
    pz = lx * e[2] + ly * e[5] + lz * e[8]
    r = jnp.sqrt(px * px + py * py + pz * pz + 1e-12)
    ct = jnp.clip(pz / r, -1.0 + 1e-6, 1.0 - 1e-6)
    xv = px + 1e-12
    rho = jnp.sqrt(xv * xv + py * py)
    cp = jnp.where(rho > 0.0, xv / rho, 1.0)
    one = jnp.ones_like(r)
    rad = [one, r, r * r]
    ang1 = [one, ct, 2.0 * ct * ct - 1.0]
    ang2 = [one, cp, 2.0 * cp * cp - 1.0]
    feats = feat_ref[...]
    fk = [b2f(feats[:, k * C_IN:(k + 1) * C_IN]) for k in range(K_NN)]
    g_cols = []
    for n in range(3):
        for l in range(3):
            for m in range(3):
                bas = b2f(rad[n] * ang1[l] * ang2[m])
                acc = bas[:, 0:1] * fk[0]
                for k in range(1, K_NN):
                    acc = acc + bas[:, k:k + 1] * fk[k]
                g_cols.append(acc)
    g = jnp.concatenate(g_cols, axis=1)
    out_ref[...] = jnp.dot(
        g.astype(jnp.bfloat16), w_ref[...].astype(jnp.bfloat16),
        preferred_element_type=jnp.float32)


def _aggregate(npos48, eig9, featg, wmat):
    return pl.pallas_call(
        _agg_body,
        grid=(N_PTS // AGG_BLK,),
        in_specs=[
            pl.BlockSpec((AGG_BLK, 3 * K_NN), lambda i: (i, 0)),
            pl.BlockSpec((AGG_BLK, 9), lambda i: (i, 0)),
            pl.BlockSpec((AGG_BLK, K_NN * C_IN), lambda i: (i, 0)),
            pl.BlockSpec((N_BASIS * C_IN, C_OUT), lambda i: (0, 0)),
        ],
        out_specs=[
            pl.BlockSpec((AGG_BLK, 3), lambda i: (i, 0)),
            pl.BlockSpec((AGG_BLK, C_OUT), lambda i: (i, 0)),
        ],
        out_shape=[
            jax.ShapeDtypeStruct((N_PTS, 3), jnp.float32),
            jax.ShapeDtypeStruct((N_PTS, C_OUT), jnp.float32),
        ],
    )(npos48, eig9, featg, wmat)


def kernel(position_matrix, channel_matrix, W):
    idx = _knn(position_matrix)
    idxf = idx.reshape(_GATHER_B)
    pos128 = jnp.pad(position_matrix, ((0, 0), (0, C_IN - 3)))
    npos = _sc_gather_rows(pos128, idxf).reshape(N_PTS, K_NN, C_IN)[:, :, :3]
    centers0 = jnp.mean(npos, axis=1)
    local0 = npos - centers0[:, None, :]
    cov = jnp.einsum('pki,pkj->pij', local0, local0) / float(K_NN)
    cov6 = jnp.stack([cov[:, 0, 0], cov[:, 0, 1], cov[:, 0, 2],
                      cov[:, 1, 1], cov[:, 1, 2], cov[:, 2, 2]], axis=0)
    eig9 = _eigh3(cov6).T
    featg = _sc_gather_rows(channel_matrix, idxf).reshape(N_PTS, K_NN * C_IN)
    npos48 = npos.transpose(0, 2, 1).reshape(N_PTS, 3 * K_NN)
    wmat = W.transpose(2, 1, 0).reshape(N_BASIS * C_IN, C_OUT)
    centers, out = _aggregate(npos48, eig9, featg, wmat)
    return centers, out
